# Initial kernel scaffold; baseline (speedup 1.0000x reference)
import jax
import jax.numpy as jnp
from jax.experimental import pallas as pl


def kernel(text, table, W, b):
    raise NotImplementedError("write your pallas kernel here")


# speedup vs baseline: 1.1709x; 1.1709x over previous
import functools

import jax
import jax.numpy as jnp
from jax import lax
from jax.experimental import pallas as pl
from jax.experimental.pallas import tpu as pltpu
from jax.experimental.pallas import tpu_sc as plsc

_NC = 2
_NS = 16
_NW = _NC * _NS
_CHUNK = 128


def _gather_body(idx_hbm, tab_hbm, out_hbm, idx_v, buf0, buf1, gs0, gs1, ws0, ws1):
    n_chunks = idx_v.shape[0]
    wid = lax.axis_index("s") * _NC + lax.axis_index("c")
    pltpu.sync_copy(idx_hbm.at[wid], idx_v)
    out_base = wid * (n_chunks * _CHUNK)

    def g_copy(buf, sem, chunk):
        return pltpu.make_async_copy(tab_hbm.at[idx_v.at[chunk]], buf, sem)

    def w_copy(buf, sem, chunk):
        return pltpu.make_async_copy(
            buf, out_hbm.at[pl.ds(out_base + chunk * _CHUNK, _CHUNK)], sem)

    g_copy(buf0, gs0, 0).start()
    g_copy(buf1, gs1, 1).start()

    def body(i, carry):
        c0 = 2 * i
        g_copy(buf0, gs0, c0).wait()
        w_copy(buf0, ws0, c0).start()
        g_copy(buf1, gs1, c0 + 1).wait()
        w_copy(buf1, ws1, c0 + 1).start()
        w_copy(buf0, ws0, c0).wait()
        g_copy(buf0, gs0, c0 + 2).start()
        w_copy(buf1, ws1, c0 + 1).wait()
        g_copy(buf1, gs1, c0 + 3).start()
        return carry

    lax.fori_loop(0, n_chunks // 2 - 1, body, 0)
    cl = n_chunks - 2
    g_copy(buf0, gs0, cl).wait()
    w_copy(buf0, ws0, cl).start()
    g_copy(buf1, gs1, cl + 1).wait()
    w_copy(buf1, ws1, cl + 1).start()
    w_copy(buf0, ws0, cl).wait()
    w_copy(buf1, ws1, cl + 1).wait()


@functools.lru_cache(maxsize=None)
def _make_gather(n_rows, n_chunks, dim):
    return functools.partial(
        pl.kernel,
        mesh=plsc.VectorSubcoreMesh(core_axis_name="c", subcore_axis_name="s"),
        out_type=jax.ShapeDtypeStruct((n_rows, dim), jnp.float32),
        scratch_types=[
            pltpu.VMEM((n_chunks, _CHUNK), jnp.int32),
            pltpu.VMEM((_CHUNK, dim), jnp.float32),
            pltpu.VMEM((_CHUNK, dim), jnp.float32),
            pltpu.SemaphoreType.DMA,
            pltpu.SemaphoreType.DMA,
            pltpu.SemaphoreType.DMA,
            pltpu.SemaphoreType.DMA,
        ],
        compiler_params=pltpu.CompilerParams(use_tc_tiling_on_sc=False),
    )(_gather_body)


def _linear_body(emb_ref, w_ref, b_ref, out_ref):
    y = lax.dot_general(emb_ref[...], w_ref[...], (((1,), (1,)), ((), ())),
                        preferred_element_type=jnp.float32)
    out_ref[...] = jnp.maximum(y + b_ref[...], 0.0)


@functools.lru_cache(maxsize=None)
def _make_linear(n_rows, dim, out_dim, bm):
    return pl.pallas_call(
        _linear_body,
        grid=(n_rows // bm,),
        in_specs=[
            pl.BlockSpec((bm, dim), lambda i: (i, 0)),
            pl.BlockSpec((out_dim, dim), lambda i: (0, 0)),
            pl.BlockSpec((1, out_dim), lambda i: (0, 0)),
        ],
        out_specs=pl.BlockSpec((bm, out_dim), lambda i: (i, 0)),
        out_shape=jax.ShapeDtypeStruct((n_rows, out_dim), jnp.float32),
        compiler_params=pltpu.CompilerParams(
            dimension_semantics=("arbitrary",)),
    )


def kernel(text, table, W, b):
    batch, hist = text.shape
    _, dim = table.shape
    out_dim = W.shape[0]
    n_rows = batch * hist
    n_chunks = n_rows // (_NW * _CHUNK)
    idx = text.reshape(_NW, n_chunks, _CHUNK)
    emb = _make_gather(n_rows, n_chunks, dim)(idx, table)
    out = _make_linear(n_rows, dim, out_dim, 8192)(emb, W, b.reshape(1, out_dim))
    return out.reshape(batch, hist, out_dim)


# speedup vs baseline: 1.5118x; 1.2911x over previous
import functools

import jax
import jax.numpy as jnp
from jax import lax
from jax.experimental import pallas as pl
from jax.experimental.pallas import tpu as pltpu
from jax.experimental.pallas import tpu_sc as plsc

_NC = 2
_NS = 16
_NW = _NC * _NS
_CHUNK = 128


def _gather_body(idx_hbm, tab_hbm, out_hbm, idx_v, buf0, buf1, gs0, gs1, ws0, ws1):
    n_chunks = idx_v.shape[0]
    dim = out_hbm.shape[1]
    wid = lax.axis_index("s") * _NC + lax.axis_index("c")
    pltpu.sync_copy(idx_hbm.at[wid], idx_v)
    out_base = wid * (n_chunks * _CHUNK)

    def g_copy(buf, sem, chunk):
        return pltpu.make_async_copy(tab_hbm.at[idx_v.at[chunk]], buf, sem)

    def w_copy(buf, sem, chunk):
        return pltpu.make_async_copy(
            buf.at[:, pl.ds(0, dim)],
            out_hbm.at[pl.ds(out_base + chunk * _CHUNK, _CHUNK)], sem)

    g_copy(buf0, gs0, 0).start()
    g_copy(buf1, gs1, 1).start()

    def body(i, carry):
        c0 = 2 * i
        g_copy(buf0, gs0, c0).wait()
        w_copy(buf0, ws0, c0).start()
        g_copy(buf1, gs1, c0 + 1).wait()
        w_copy(buf1, ws1, c0 + 1).start()
        w_copy(buf0, ws0, c0).wait()
        g_copy(buf0, gs0, c0 + 2).start()
        w_copy(buf1, ws1, c0 + 1).wait()
        g_copy(buf1, gs1, c0 + 3).start()
        return carry

    lax.fori_loop(0, n_chunks // 2 - 1, body, 0)
    cl = n_chunks - 2
    g_copy(buf0, gs0, cl).wait()
    w_copy(buf0, ws0, cl).start()
    g_copy(buf1, gs1, cl + 1).wait()
    w_copy(buf1, ws1, cl + 1).start()
    w_copy(buf0, ws0, cl).wait()
    w_copy(buf1, ws1, cl + 1).wait()


@functools.lru_cache(maxsize=None)
def _make_gather(n_rows, n_chunks, dim):
    return functools.partial(
        pl.kernel,
        mesh=plsc.VectorSubcoreMesh(core_axis_name="c", subcore_axis_name="s"),
        out_type=jax.ShapeDtypeStruct((n_rows, dim), jnp.float32),
        scratch_types=[
            pltpu.VMEM((n_chunks, _CHUNK), jnp.int32),
            pltpu.VMEM((_CHUNK, 2 * dim), jnp.float32),
            pltpu.VMEM((_CHUNK, 2 * dim), jnp.float32),
            pltpu.SemaphoreType.DMA,
            pltpu.SemaphoreType.DMA,
            pltpu.SemaphoreType.DMA,
            pltpu.SemaphoreType.DMA,
        ],
        compiler_params=pltpu.CompilerParams(use_tc_tiling_on_sc=False),
    )(_gather_body)


def _transform_body(tab_ref, w_ref, b_ref, out_ref):
    x = tab_ref[...]
    y = lax.dot_general(x, w_ref[...], (((1,), (1,)), ((), ())),
                        preferred_element_type=jnp.float32)
    y = jnp.maximum(y + b_ref[...], 0.0)
    out_ref[...] = jnp.concatenate([y, y], axis=1)


@functools.lru_cache(maxsize=None)
def _make_transform(vocab, dim, out_dim, bm):
    return pl.pallas_call(
        _transform_body,
        grid=(vocab // bm,),
        in_specs=[
            pl.BlockSpec((bm, dim), lambda i: (i, 0)),
            pl.BlockSpec((out_dim, dim), lambda i: (0, 0)),
            pl.BlockSpec((1, out_dim), lambda i: (0, 0)),
        ],
        out_specs=pl.BlockSpec((bm, 2 * out_dim), lambda i: (i, 0)),
        out_shape=jax.ShapeDtypeStruct((vocab, 2 * out_dim), jnp.float32),
        compiler_params=pltpu.CompilerParams(
            dimension_semantics=("arbitrary",)),
    )


def kernel(text, table, W, b):
    batch, hist = text.shape
    vocab, dim = table.shape
    out_dim = W.shape[0]
    n_rows = batch * hist
    n_chunks = n_rows // (_NW * _CHUNK)
    idx = text.reshape(_NW, n_chunks, _CHUNK)
    t2 = _make_transform(vocab, dim, out_dim, 8000)(table, W, b.reshape(1, out_dim))
    out = _make_gather(n_rows, n_chunks, out_dim)(idx, t2)
    return out.reshape(batch, hist, out_dim)


# speedup vs baseline: 1.5727x; 1.0403x over previous
import functools

import jax
import jax.numpy as jnp
from jax import lax
from jax.experimental import pallas as pl
from jax.experimental.pallas import tpu as pltpu
from jax.experimental.pallas import tpu_sc as plsc

_NC = 2
_NS = 16
_NW = _NC * _NS
_CHUNK = 128


def _gather_body(idx_hbm, tab_hbm, out_hbm, idx_v, buf0, buf1, gs0, gs1, ws0, ws1):
    n_chunks = idx_v.shape[0]
    dim = tab_hbm.shape[1]
    half_rows = out_hbm.shape[0]
    wid = lax.axis_index("s") * _NC + lax.axis_index("c")
    pltpu.sync_copy(idx_hbm.at[wid], idx_v)
    flat0 = wid * (n_chunks * _CHUNK)
    col0 = (flat0 // half_rows) * dim
    row0 = flat0 % half_rows

    def g_copy(buf, sem, chunk):
        return pltpu.make_async_copy(tab_hbm.at[idx_v.at[chunk]], buf, sem)

    def w_copy(buf, sem, chunk):
        return pltpu.make_async_copy(
            buf,
            out_hbm.at[pl.ds(row0 + chunk * _CHUNK, _CHUNK), pl.ds(col0, dim)],
            sem)

    g_copy(buf0, gs0, 0).start()
    g_copy(buf1, gs1, 1).start()

    def body(i, carry):
        c0 = 2 * i
        g_copy(buf0, gs0, c0).wait()
        w_copy(buf0, ws0, c0).start()
        g_copy(buf1, gs1, c0 + 1).wait()
        w_copy(buf1, ws1, c0 + 1).start()
        w_copy(buf0, ws0, c0).wait()
        g_copy(buf0, gs0, c0 + 2).start()
        w_copy(buf1, ws1, c0 + 1).wait()
        g_copy(buf1, gs1, c0 + 3).start()
        return carry

    lax.fori_loop(0, n_chunks // 2 - 1, body, 0)
    cl = n_chunks - 2
    g_copy(buf0, gs0, cl).wait()
    w_copy(buf0, ws0, cl).start()
    g_copy(buf1, gs1, cl + 1).wait()
    w_copy(buf1, ws1, cl + 1).start()
    w_copy(buf0, ws0, cl).wait()
    w_copy(buf1, ws1, cl + 1).wait()


@functools.lru_cache(maxsize=None)
def _make_gather(half_rows, n_chunks, dim):
    return functools.partial(
        pl.kernel,
        mesh=plsc.VectorSubcoreMesh(core_axis_name="c", subcore_axis_name="s"),
        out_type=jax.ShapeDtypeStruct((half_rows, 2 * dim), jnp.float32),
        scratch_types=[
            pltpu.VMEM((n_chunks, _CHUNK), jnp.int32),
            pltpu.VMEM((_CHUNK, dim), jnp.float32),
            pltpu.VMEM((_CHUNK, dim), jnp.float32),
            pltpu.SemaphoreType.DMA,
            pltpu.SemaphoreType.DMA,
            pltpu.SemaphoreType.DMA,
            pltpu.SemaphoreType.DMA,
        ],
        compiler_params=pltpu.CompilerParams(use_tc_tiling_on_sc=False),
    )(_gather_body)


def _linear_body(emb_ref, w_ref, b_ref, out_ref):
    j = pl.program_id(1)
    x = emb_ref[...]
    wt = w_ref[...]
    m0 = (j == 0).astype(jnp.float32)
    w2 = jnp.concatenate([wt * m0, wt * (1.0 - m0)], axis=1)
    y = lax.dot_general(x, w2, (((1,), (1,)), ((), ())),
                        preferred_element_type=jnp.float32)
    y = jnp.maximum(y + b_ref[...], 0.0)
    out_ref[...] = y.reshape(out_ref.shape)


@functools.lru_cache(maxsize=None)
def _make_linear(batch, hist, dim, out_dim, nb):
    half_batch = batch // 2
    return pl.pallas_call(
        _linear_body,
        grid=(half_batch // nb, 2),
        in_specs=[
            pl.BlockSpec((nb * hist, 2 * dim), lambda i, j: (i, 0)),
            pl.BlockSpec((out_dim, dim), lambda i, j: (0, 0)),
            pl.BlockSpec((1, out_dim), lambda i, j: (0, 0)),
        ],
        out_specs=pl.BlockSpec((nb, hist, out_dim),
                               lambda i, j: (j * (half_batch // nb) + i, 0, 0)),
        out_shape=jax.ShapeDtypeStruct((batch, hist, out_dim), jnp.float32),
        compiler_params=pltpu.CompilerParams(
            dimension_semantics=("arbitrary", "arbitrary")),
    )


def kernel(text, table, W, b):
    batch, hist = text.shape
    vocab, dim = table.shape
    out_dim = W.shape[0]
    n_rows = batch * hist
    n_chunks = n_rows // (_NW * _CHUNK)
    idx = text.reshape(_NW, n_chunks, _CHUNK)
    emb = _make_gather(n_rows // 2, n_chunks, dim)(idx, table)
    return _make_linear(batch, hist, dim, out_dim, 128)(
        emb, W, b.reshape(1, out_dim))


# speedup vs baseline: 3.3739x; 2.1453x over previous
import functools

import jax
import jax.numpy as jnp
from jax import lax
from jax.experimental import pallas as pl
from jax.experimental.pallas import tpu as pltpu
from jax.experimental.pallas import tpu_sc as plsc

_NC = 2
_NS = 16
_NW = _NC * _NS
_CHUNK = 128


def _relayout_body(tabt_ref, out_ref):
    dim, bm = tabt_ref.shape
    xt = tabt_ref[...].T
    out_ref[:, pl.ds(0, dim)] = xt[: bm // 2]
    out_ref[:, pl.ds(dim, dim)] = xt[bm // 2:]


@functools.lru_cache(maxsize=None)
def _make_relayout(vocab, dim, bm):
    return pl.pallas_call(
        _relayout_body,
        grid=((vocab + bm - 1) // bm,),
        in_specs=[pl.BlockSpec((dim, bm), lambda i: (0, i))],
        out_specs=pl.BlockSpec((bm // 2, 2 * dim), lambda i: (i, 0)),
        out_shape=jax.ShapeDtypeStruct((vocab // 2, 2 * dim), jnp.float32),
        compiler_params=pltpu.CompilerParams(
            dimension_semantics=("arbitrary",)),
    )


def _gather_body(idx_hbm, tab_hbm, out_hbm, idx_v, buf0, buf1, gs0, gs1, ws0, ws1):
    n_chunks = idx_v.shape[0]
    dim = tab_hbm.shape[1]
    half_batch = out_hbm.shape[1]
    batch = 2 * half_batch
    wid = lax.axis_index("s") * _NC + lax.axis_index("c")
    pltpu.sync_copy(idx_hbm.at[wid], idx_v)
    flat0 = wid * (n_chunks * _CHUNK)

    def remap(t, carry):
        c = t // (_CHUNK // 16)
        k = t % (_CHUNK // 16)
        v = idx_v[c, pl.ds(k * 16, 16)]
        q = lax.bitwise_and(v, 8191)
        base = lax.bitwise_and(v, ~8191)
        p = base + 2 * q - jnp.where(q < 4096, 0, 8191)
        idx_v[c, pl.ds(k * 16, 16)] = p
        return carry

    lax.fori_loop(0, n_chunks * (_CHUNK // 16), remap, 0)

    def g_copy(buf, sem, chunk):
        return pltpu.make_async_copy(tab_hbm.at[idx_v.at[chunk]], buf, sem)

    def w_copy(buf, sem, chunk):
        fl = flat0 + chunk * _CHUNK
        l = fl // batch
        bb = fl % batch
        dst = out_hbm.at[l, pl.ds(bb % half_batch, _CHUNK),
                         pl.ds((bb // half_batch) * dim, dim)]
        return pltpu.make_async_copy(buf, dst, sem)

    g_copy(buf0, gs0, 0).start()
    g_copy(buf1, gs1, 1).start()

    def body(i, carry):
        c0 = 2 * i
        g_copy(buf0, gs0, c0).wait()
        w_copy(buf0, ws0, c0).start()
        g_copy(buf1, gs1, c0 + 1).wait()
        w_copy(buf1, ws1, c0 + 1).start()
        w_copy(buf0, ws0, c0).wait()
        g_copy(buf0, gs0, c0 + 2).start()
        w_copy(buf1, ws1, c0 + 1).wait()
        g_copy(buf1, gs1, c0 + 3).start()
        return carry

    lax.fori_loop(0, n_chunks // 2 - 1, body, 0)
    cl = n_chunks - 2
    g_copy(buf0, gs0, cl).wait()
    w_copy(buf0, ws0, cl).start()
    g_copy(buf1, gs1, cl + 1).wait()
    w_copy(buf1, ws1, cl + 1).start()
    w_copy(buf0, ws0, cl).wait()
    w_copy(buf1, ws1, cl + 1).wait()


@functools.lru_cache(maxsize=None)
def _make_gather(hist, half_batch, n_chunks, dim):
    return functools.partial(
        pl.kernel,
        mesh=plsc.VectorSubcoreMesh(core_axis_name="c", subcore_axis_name="s"),
        out_type=jax.ShapeDtypeStruct((hist, half_batch, 2 * dim), jnp.float32),
        scratch_types=[
            pltpu.VMEM((n_chunks, _CHUNK), jnp.int32),
            pltpu.VMEM((_CHUNK, dim), jnp.float32),
            pltpu.VMEM((_CHUNK, dim), jnp.float32),
            pltpu.SemaphoreType.DMA,
            pltpu.SemaphoreType.DMA,
            pltpu.SemaphoreType.DMA,
            pltpu.SemaphoreType.DMA,
        ],
        compiler_params=pltpu.CompilerParams(use_tc_tiling_on_sc=False),
    )(_gather_body)


def _linear_body(emb_ref, w_ref, b_ref, out_ref):
    j = pl.program_id(1)
    hist, dim = emb_ref.shape[0], w_ref.shape[1]
    wt = w_ref[...]
    m0 = (j == 0).astype(jnp.float32)
    w2 = jnp.concatenate([wt * m0, wt * (1.0 - m0)], axis=1)
    bcol = b_ref[...]
    for l in range(hist):
        y = lax.dot_general(w2, emb_ref[l], (((1,), (1,)), ((), ())),
                            preferred_element_type=jnp.float32)
        out_ref[pl.ds(l * dim, dim), :] = jnp.maximum(y + bcol, 0.0)


@functools.lru_cache(maxsize=None)
def _make_linear(batch, hist, dim, out_dim, nbb):
    half_batch = batch // 2
    nblk = half_batch // nbb
    return pl.pallas_call(
        _linear_body,
        grid=(nblk, 2),
        in_specs=[
            pl.BlockSpec((hist, nbb, 2 * dim), lambda i, j: (0, i, 0)),
            pl.BlockSpec((out_dim, dim), lambda i, j: (0, 0)),
            pl.BlockSpec((out_dim, 1), lambda i, j: (0, 0)),
        ],
        out_specs=pl.BlockSpec((hist * out_dim, nbb),
                               lambda i, j: (0, j * nblk + i)),
        out_shape=jax.ShapeDtypeStruct((hist * out_dim, batch), jnp.float32),
        compiler_params=pltpu.CompilerParams(
            dimension_semantics=("arbitrary", "arbitrary")),
    )


def kernel(text, table, W, b):
    batch, hist = text.shape
    vocab, dim = table.shape
    out_dim = W.shape[0]
    n_rows = batch * hist
    n_chunks = n_rows // (_NW * _CHUNK)
    idx = text.T.reshape(_NW, n_chunks, _CHUNK)
    tab_pairs = _make_relayout(vocab, dim, 8192)(table.T)
    tab_lin = tab_pairs.reshape(vocab, dim)
    emb = _make_gather(hist, batch // 2, n_chunks, dim)(idx, tab_lin)
    out2 = _make_linear(batch, hist, dim, out_dim, 256)(
        emb, W, b.reshape(out_dim, 1))
    return out2.reshape(hist, out_dim, batch).transpose(2, 0, 1)


# speedup vs baseline: 3.3894x; 1.0046x over previous
import functools

import jax
import jax.numpy as jnp
from jax import lax
from jax.experimental import pallas as pl
from jax.experimental.pallas import tpu as pltpu
from jax.experimental.pallas import tpu_sc as plsc

_NC = 2
_NS = 16
_NW = _NC * _NS
_CHUNK = 128


def _relayout_body(tail_half, tabt_ref, out_ref):
    dim, bm = tabt_ref.shape
    half = bm // 2
    i = pl.program_id(0)
    nblk = pl.num_programs(0)
    xt = tabt_ref[...].T
    hi = jnp.where(i == nblk - 1, xt[tail_half:tail_half + half],
                   xt[half:])
    out_ref[:, pl.ds(0, dim)] = xt[:half]
    out_ref[:, pl.ds(dim, dim)] = hi


@functools.lru_cache(maxsize=None)
def _make_relayout(vocab, dim, bm):
    tail = vocab % bm
    tail_half = (tail // 2) if tail else (bm // 2)
    return pl.pallas_call(
        functools.partial(_relayout_body, tail_half),
        grid=((vocab + bm - 1) // bm,),
        in_specs=[pl.BlockSpec((dim, bm), lambda i: (0, i))],
        out_specs=pl.BlockSpec((bm // 2, 2 * dim), lambda i: (i, 0)),
        out_shape=jax.ShapeDtypeStruct((vocab // 2, 2 * dim), jnp.float32),
        compiler_params=pltpu.CompilerParams(
            dimension_semantics=("arbitrary",)),
    )


def _gather_body(idx_hbm, tab_hbm, out_hbm, idx_v, buf0, buf1, gs0, gs1, ws0, ws1):
    n_chunks = idx_v.shape[0]
    dim = tab_hbm.shape[1]
    half_batch = out_hbm.shape[1]
    batch = 2 * half_batch
    wid = lax.axis_index("s") * _NC + lax.axis_index("c")
    pltpu.sync_copy(idx_hbm.at[wid], idx_v)
    flat0 = wid * (n_chunks * _CHUNK)

    def g_copy(buf, sem, chunk):
        return pltpu.make_async_copy(tab_hbm.at[idx_v.at[chunk]], buf, sem)

    def w_copy(buf, sem, chunk):
        fl = flat0 + chunk * _CHUNK
        l = fl // batch
        bb = fl % batch
        dst = out_hbm.at[l, pl.ds(bb % half_batch, _CHUNK),
                         pl.ds((bb // half_batch) * dim, dim)]
        return pltpu.make_async_copy(buf, dst, sem)

    g_copy(buf0, gs0, 0).start()
    g_copy(buf1, gs1, 1).start()

    def body(i, carry):
        c0 = 2 * i
        g_copy(buf0, gs0, c0).wait()
        w_copy(buf0, ws0, c0).start()
        g_copy(buf1, gs1, c0 + 1).wait()
        w_copy(buf1, ws1, c0 + 1).start()
        w_copy(buf0, ws0, c0).wait()
        g_copy(buf0, gs0, c0 + 2).start()
        w_copy(buf1, ws1, c0 + 1).wait()
        g_copy(buf1, gs1, c0 + 3).start()
        return carry

    lax.fori_loop(0, n_chunks // 2 - 1, body, 0)
    cl = n_chunks - 2
    g_copy(buf0, gs0, cl).wait()
    w_copy(buf0, ws0, cl).start()
    g_copy(buf1, gs1, cl + 1).wait()
    w_copy(buf1, ws1, cl + 1).start()
    w_copy(buf0, ws0, cl).wait()
    w_copy(buf1, ws1, cl + 1).wait()


@functools.lru_cache(maxsize=None)
def _make_gather(hist, half_batch, n_chunks, dim):
    return functools.partial(
        pl.kernel,
        mesh=plsc.VectorSubcoreMesh(core_axis_name="c", subcore_axis_name="s"),
        out_type=jax.ShapeDtypeStruct((hist, half_batch, 2 * dim), jnp.float32),
        scratch_types=[
            pltpu.VMEM((n_chunks, _CHUNK), jnp.int32),
            pltpu.VMEM((_CHUNK, dim), jnp.float32),
            pltpu.VMEM((_CHUNK, dim), jnp.float32),
            pltpu.SemaphoreType.DMA,
            pltpu.SemaphoreType.DMA,
            pltpu.SemaphoreType.DMA,
            pltpu.SemaphoreType.DMA,
        ],
        compiler_params=pltpu.CompilerParams(use_tc_tiling_on_sc=False),
    )(_gather_body)


def _linear_body(emb_ref, w_ref, b_ref, out_ref):
    j = pl.program_id(1)
    hist, dim = emb_ref.shape[0], w_ref.shape[1]
    wt = w_ref[...]
    m0 = (j == 0).astype(jnp.float32)
    w2 = jnp.concatenate([wt * m0, wt * (1.0 - m0)], axis=1)
    bcol = b_ref[...]
    for l in range(hist):
        y = lax.dot_general(w2, emb_ref[l], (((1,), (1,)), ((), ())),
                            preferred_element_type=jnp.float32)
        out_ref[pl.ds(l * dim, dim), :] = jnp.maximum(y + bcol, 0.0)


@functools.lru_cache(maxsize=None)
def _make_linear(batch, hist, dim, out_dim, nbb):
    half_batch = batch // 2
    nblk = half_batch // nbb
    return pl.pallas_call(
        _linear_body,
        grid=(nblk, 2),
        in_specs=[
            pl.BlockSpec((hist, nbb, 2 * dim), lambda i, j: (0, i, 0)),
            pl.BlockSpec((out_dim, dim), lambda i, j: (0, 0)),
            pl.BlockSpec((out_dim, 1), lambda i, j: (0, 0)),
        ],
        out_specs=pl.BlockSpec((hist * out_dim, nbb),
                               lambda i, j: (0, j * nblk + i)),
        out_shape=jax.ShapeDtypeStruct((hist * out_dim, batch), jnp.float32),
        compiler_params=pltpu.CompilerParams(
            dimension_semantics=("arbitrary", "arbitrary")),
    )


def kernel(text, table, W, b):
    batch, hist = text.shape
    vocab, dim = table.shape
    out_dim = W.shape[0]
    n_rows = batch * hist
    n_chunks = n_rows // (_NW * _CHUNK)
    bm = 8192
    full = (vocab // bm) * bm
    ht = max((vocab - full) // 2, 1)
    q = jnp.bitwise_and(text, bm - 1)
    pseudo_full = jnp.bitwise_and(text, ~jnp.int32(bm - 1)) + 2 * q \
        - jnp.where(q < bm // 2, 0, bm - 1).astype(jnp.int32)
    qt = text - full
    pseudo_tail = full + 2 * qt \
        - jnp.where(qt < ht, 0, 2 * ht - 1).astype(jnp.int32)
    pseudo = jnp.where(text < full, pseudo_full, pseudo_tail)
    idx = pseudo.T.reshape(_NW, n_chunks, _CHUNK)
    tab_pairs = _make_relayout(vocab, dim, 8192)(table.T)
    tab_lin = tab_pairs.reshape(vocab, dim)
    emb = _make_gather(hist, batch // 2, n_chunks, dim)(idx, tab_lin)
    out2 = _make_linear(batch, hist, dim, out_dim, 256)(
        emb, W, b.reshape(out_dim, 1))
    return out2.reshape(hist, out_dim, batch).transpose(2, 0, 1)


# speedup vs baseline: 3.7019x; 1.0922x over previous
import functools

import jax
import jax.numpy as jnp
from jax import lax
from jax.experimental import pallas as pl
from jax.experimental.pallas import tpu as pltpu
from jax.experimental.pallas import tpu_sc as plsc

_NC = 2
_NS = 16
_NW = _NC * _NS
_CHUNK = 128


def _relayout_body(tail_half, tabt_ref, out_ref):
    dim, bm = tabt_ref.shape
    half = bm // 2
    i = pl.program_id(0)
    nblk = pl.num_programs(0)
    xt = tabt_ref[...].T
    hi = jnp.where(i == nblk - 1, xt[tail_half:tail_half + half],
                   xt[half:])
    out_ref[:, pl.ds(0, dim)] = xt[:half]
    out_ref[:, pl.ds(dim, dim)] = hi


@functools.lru_cache(maxsize=None)
def _make_relayout(vocab, dim, bm):
    tail = vocab % bm
    tail_half = (tail // 2) if tail else (bm // 2)
    return pl.pallas_call(
        functools.partial(_relayout_body, tail_half),
        grid=((vocab + bm - 1) // bm,),
        in_specs=[pl.BlockSpec((dim, bm), lambda i: (0, i))],
        out_specs=pl.BlockSpec((bm // 2, 2 * dim), lambda i: (i, 0)),
        out_shape=jax.ShapeDtypeStruct((vocab // 2, 2 * dim), jnp.float32),
        compiler_params=pltpu.CompilerParams(
            dimension_semantics=("arbitrary",)),
    )


def _gather_body(idx_hbm, tab_hbm, out_hbm, idx_v, buf0, buf1, gs0, gs1, ws0, ws1):
    n_chunks = idx_v.shape[0]
    dim = tab_hbm.shape[1]
    half_batch = out_hbm.shape[1]
    batch = 2 * half_batch
    wid = lax.axis_index("s") * _NC + lax.axis_index("c")
    pltpu.sync_copy(idx_hbm.at[wid], idx_v)
    flat0 = wid * (n_chunks * _CHUNK)

    def g_copy(buf, sem, chunk):
        return pltpu.make_async_copy(tab_hbm.at[idx_v.at[chunk]], buf, sem)

    def w_copy(buf, sem, chunk):
        fl = flat0 + chunk * _CHUNK
        l = fl // batch
        bb = fl % batch
        dst = out_hbm.at[l, pl.ds(bb % half_batch, _CHUNK),
                         pl.ds((bb // half_batch) * dim, dim)]
        return pltpu.make_async_copy(buf, dst, sem)

    g_copy(buf0, gs0, 0).start()
    g_copy(buf1, gs1, 1).start()

    def body(i, carry):
        c0 = 2 * i
        g_copy(buf0, gs0, c0).wait()
        w_copy(buf0, ws0, c0).start()
        g_copy(buf1, gs1, c0 + 1).wait()
        w_copy(buf1, ws1, c0 + 1).start()
        w_copy(buf0, ws0, c0).wait()
        g_copy(buf0, gs0, c0 + 2).start()
        w_copy(buf1, ws1, c0 + 1).wait()
        g_copy(buf1, gs1, c0 + 3).start()
        return carry

    lax.fori_loop(0, n_chunks // 2 - 1, body, 0)
    cl = n_chunks - 2
    g_copy(buf0, gs0, cl).wait()
    w_copy(buf0, ws0, cl).start()
    g_copy(buf1, gs1, cl + 1).wait()
    w_copy(buf1, ws1, cl + 1).start()
    w_copy(buf0, ws0, cl).wait()
    w_copy(buf1, ws1, cl + 1).wait()


@functools.lru_cache(maxsize=None)
def _make_gather(hist, half_batch, n_chunks, dim):
    return functools.partial(
        pl.kernel,
        mesh=plsc.VectorSubcoreMesh(core_axis_name="c", subcore_axis_name="s"),
        out_type=jax.ShapeDtypeStruct((hist, half_batch, 2 * dim), jnp.float32),
        scratch_types=[
            pltpu.VMEM((n_chunks, _CHUNK), jnp.int32),
            pltpu.VMEM((_CHUNK, dim), jnp.float32),
            pltpu.VMEM((_CHUNK, dim), jnp.float32),
            pltpu.SemaphoreType.DMA,
            pltpu.SemaphoreType.DMA,
            pltpu.SemaphoreType.DMA,
            pltpu.SemaphoreType.DMA,
        ],
        compiler_params=pltpu.CompilerParams(use_tc_tiling_on_sc=False),
    )(_gather_body)


def _linear_body(emb_ref, w_ref, b_ref, out_ref):
    j = pl.program_id(1)
    hist, dim = emb_ref.shape[0], w_ref.shape[1]
    wt = w_ref[...]
    m0 = (j == 0).astype(jnp.float32)
    w2 = jnp.concatenate([wt * m0, wt * (1.0 - m0)], axis=1)
    bcol = b_ref[...]
    for l in range(hist):
        y = lax.dot_general(w2, emb_ref[l], (((1,), (1,)), ((), ())),
                            preferred_element_type=jnp.float32)
        out_ref[pl.ds(l * dim, dim), :] = jnp.maximum(y + bcol, 0.0)


@functools.lru_cache(maxsize=None)
def _make_linear(batch, hist, dim, out_dim, nbb):
    half_batch = batch // 2
    nblk = half_batch // nbb
    return pl.pallas_call(
        _linear_body,
        grid=(nblk, 2),
        in_specs=[
            pl.BlockSpec((hist, nbb, 2 * dim), lambda i, j: (0, i, 0)),
            pl.BlockSpec((out_dim, dim), lambda i, j: (0, 0)),
            pl.BlockSpec((out_dim, 1), lambda i, j: (0, 0)),
        ],
        out_specs=pl.BlockSpec((hist * out_dim, nbb),
                               lambda i, j: (0, j * nblk + i)),
        out_shape=jax.ShapeDtypeStruct((hist * out_dim, batch), jnp.float32),
        compiler_params=pltpu.CompilerParams(
            dimension_semantics=("arbitrary", "arbitrary")),
    )


def kernel(text, table, W, b):
    batch, hist = text.shape
    vocab, dim = table.shape
    out_dim = W.shape[0]
    n_rows = batch * hist
    n_chunks = n_rows // (_NW * _CHUNK)
    bm = 16384
    full = (vocab // bm) * bm
    ht = max((vocab - full) // 2, 1)
    q = jnp.bitwise_and(text, bm - 1)
    pseudo_full = jnp.bitwise_and(text, ~jnp.int32(bm - 1)) + 2 * q \
        - jnp.where(q < bm // 2, 0, bm - 1).astype(jnp.int32)
    qt = text - full
    pseudo_tail = full + 2 * qt \
        - jnp.where(qt < ht, 0, 2 * ht - 1).astype(jnp.int32)
    pseudo = jnp.where(text < full, pseudo_full, pseudo_tail)
    idx = pseudo.T.reshape(_NW, n_chunks, _CHUNK)
    tab_pairs = _make_relayout(vocab, dim, 16384)(table.T)
    tab_lin = tab_pairs.reshape(vocab, dim)
    emb = _make_gather(hist, batch // 2, n_chunks, dim)(idx, tab_lin)
    out2 = _make_linear(batch, hist, dim, out_dim, 512)(
        emb, W, b.reshape(out_dim, 1))
    return out2.reshape(hist, out_dim, batch).transpose(2, 0, 1)


# speedup vs baseline: 3.9123x; 1.0568x over previous
import functools

import jax
import jax.numpy as jnp
from jax import lax
from jax.experimental import pallas as pl
from jax.experimental.pallas import tpu as pltpu
from jax.experimental.pallas import tpu_sc as plsc

_NC = 2
_NS = 16
_NW = _NC * _NS
_CHUNK = 128


def _relayout_body(tail_half, tabt_ref, out_ref):
    dim, bm = tabt_ref.shape
    half = bm // 2
    i = pl.program_id(0)
    nblk = pl.num_programs(0)
    xt = tabt_ref[...].T
    hi = jnp.where(i == nblk - 1, xt[tail_half:tail_half + half],
                   xt[half:])
    out_ref[:, pl.ds(0, dim)] = xt[:half]
    out_ref[:, pl.ds(dim, dim)] = hi


@functools.lru_cache(maxsize=None)
def _make_relayout(vocab, dim, bm):
    tail = vocab % bm
    tail_half = (tail // 2) if tail else (bm // 2)
    return pl.pallas_call(
        functools.partial(_relayout_body, tail_half),
        grid=((vocab + bm - 1) // bm,),
        in_specs=[pl.BlockSpec((dim, bm), lambda i: (0, i))],
        out_specs=pl.BlockSpec((bm // 2, 2 * dim), lambda i: (i, 0)),
        out_shape=jax.ShapeDtypeStruct((vocab // 2, 2 * dim), jnp.float32),
        compiler_params=pltpu.CompilerParams(
            dimension_semantics=("arbitrary",)),
    )


def _gather_body(idx_hbm, tab_hbm, out_hbm, idx_v, buf0, buf1, gs0, gs1, ws0, ws1):
    n_chunks = idx_v.shape[0]
    dim = tab_hbm.shape[1]
    half_batch = out_hbm.shape[1]
    batch = 2 * half_batch
    wid = lax.axis_index("s") * _NC + lax.axis_index("c")
    pltpu.sync_copy(idx_hbm.at[wid], idx_v)
    flat0 = wid * (n_chunks * _CHUNK)

    def g_copy(buf, sem, chunk):
        return pltpu.make_async_copy(tab_hbm.at[idx_v.at[chunk]], buf, sem)

    def w_copy(buf, sem, chunk):
        fl = flat0 + chunk * _CHUNK
        l = fl // batch
        bb = fl % batch
        dst = out_hbm.at[l, pl.ds(bb % half_batch, _CHUNK),
                         pl.ds((bb // half_batch) * dim, dim)]
        return pltpu.make_async_copy(buf, dst, sem)

    g_copy(buf0, gs0, 0).start()
    g_copy(buf1, gs1, 1).start()

    def body(i, carry):
        c0 = 2 * i
        g_copy(buf0, gs0, c0).wait()
        w_copy(buf0, ws0, c0).start()
        g_copy(buf1, gs1, c0 + 1).wait()
        w_copy(buf1, ws1, c0 + 1).start()
        w_copy(buf0, ws0, c0).wait()
        g_copy(buf0, gs0, c0 + 2).start()
        w_copy(buf1, ws1, c0 + 1).wait()
        g_copy(buf1, gs1, c0 + 3).start()
        return carry

    lax.fori_loop(0, n_chunks // 2 - 1, body, 0)
    cl = n_chunks - 2
    g_copy(buf0, gs0, cl).wait()
    w_copy(buf0, ws0, cl).start()
    g_copy(buf1, gs1, cl + 1).wait()
    w_copy(buf1, ws1, cl + 1).start()
    w_copy(buf0, ws0, cl).wait()
    w_copy(buf1, ws1, cl + 1).wait()


@functools.lru_cache(maxsize=None)
def _make_gather(hist, half_batch, n_chunks, dim):
    return functools.partial(
        pl.kernel,
        mesh=plsc.VectorSubcoreMesh(core_axis_name="c", subcore_axis_name="s"),
        out_type=jax.ShapeDtypeStruct((hist, half_batch, 2 * dim), jnp.float32),
        scratch_types=[
            pltpu.VMEM((n_chunks, _CHUNK), jnp.int32),
            pltpu.VMEM((_CHUNK, dim), jnp.float32),
            pltpu.VMEM((_CHUNK, dim), jnp.float32),
            pltpu.SemaphoreType.DMA,
            pltpu.SemaphoreType.DMA,
            pltpu.SemaphoreType.DMA,
            pltpu.SemaphoreType.DMA,
        ],
        compiler_params=pltpu.CompilerParams(use_tc_tiling_on_sc=False),
    )(_gather_body)


def _linear_body(*refs):
    if len(refs) == 5:
        emb_ref, w_ref, b_ref, _, out_ref = refs
    else:
        emb_ref, w_ref, b_ref, out_ref = refs
    j = pl.program_id(1)
    hist, dim = emb_ref.shape[0], w_ref.shape[1]
    wt = w_ref[...]
    m0 = (j == 0).astype(jnp.float32)
    w2 = jnp.concatenate([wt * m0, wt * (1.0 - m0)], axis=1)
    bcol = b_ref[...]
    for l in range(hist):
        y = lax.dot_general(w2, emb_ref[l], (((1,), (1,)), ((), ())),
                            preferred_element_type=jnp.float32)
        out_ref[pl.ds(l * dim, dim), :] = jnp.maximum(y + bcol, 0.0)


@functools.lru_cache(maxsize=None)
def _make_linear(batch, hist, dim, out_dim, nbb, span, col_blk0, aliased):
    nblk = (span // 2) // nbb
    in_specs = [
        pl.BlockSpec((hist, nbb, 2 * dim), lambda i, j: (0, i, 0)),
        pl.BlockSpec((out_dim, dim), lambda i, j: (0, 0)),
        pl.BlockSpec((out_dim, 1), lambda i, j: (0, 0)),
    ]
    if aliased:
        in_specs.append(pl.BlockSpec((8, 128), lambda i, j: (0, 0)))
    return pl.pallas_call(
        _linear_body,
        grid=(nblk, 2),
        in_specs=in_specs,
        out_specs=pl.BlockSpec((hist * out_dim, nbb),
                               lambda i, j: (0, col_blk0 + j * nblk + i)),
        out_shape=jax.ShapeDtypeStruct((hist * out_dim, batch), jnp.float32),
        input_output_aliases={3: 0} if aliased else {},
        compiler_params=pltpu.CompilerParams(
            dimension_semantics=("arbitrary", "arbitrary")),
    )


def kernel(text, table, W, b):
    batch, hist = text.shape
    vocab, dim = table.shape
    out_dim = W.shape[0]
    n_rows = batch * hist
    n_chunks = n_rows // (_NW * _CHUNK)
    bm = 16384
    full = (vocab // bm) * bm
    ht = max((vocab - full) // 2, 1)
    q = jnp.bitwise_and(text, bm - 1)
    pseudo_full = jnp.bitwise_and(text, ~jnp.int32(bm - 1)) + 2 * q \
        - jnp.where(q < bm // 2, 0, bm - 1).astype(jnp.int32)
    qt = text - full
    pseudo_tail = full + 2 * qt \
        - jnp.where(qt < ht, 0, 2 * ht - 1).astype(jnp.int32)
    pseudo = jnp.where(text < full, pseudo_full, pseudo_tail)
    pseudo_t = pseudo.T.reshape(hist, 2, batch // 2)
    half = batch // 2
    nc2 = n_chunks // 2
    nbb = 512
    bcol = b.reshape(out_dim, 1)
    tab_pairs = _make_relayout(vocab, dim, 16384)(table.T)
    tab_lin = tab_pairs.reshape(vocab, dim)
    idx_a = pseudo_t[:, 0].reshape(_NW, nc2, _CHUNK)
    idx_b = pseudo_t[:, 1].reshape(_NW, nc2, _CHUNK)
    emb_a = _make_gather(hist, half // 2, nc2, dim)(idx_a, tab_lin)
    out2a = _make_linear(batch, hist, dim, out_dim, nbb, half, 0, False)(
        emb_a, W, bcol)
    emb_b = _make_gather(hist, half // 2, nc2, dim)(idx_b, tab_lin)
    out2 = _make_linear(batch, hist, dim, out_dim, nbb, half,
                        half // nbb, True)(emb_b, W, bcol, out2a)
    return out2.reshape(hist, out_dim, batch).transpose(2, 0, 1)


# speedup vs baseline: 4.0190x; 1.0273x over previous
import functools

import jax
import jax.numpy as jnp
from jax import lax
from jax.experimental import pallas as pl
from jax.experimental.pallas import tpu as pltpu
from jax.experimental.pallas import tpu_sc as plsc

_NC = 2
_NS = 16
_NW = _NC * _NS
_CHUNK = 128


def _relayout_body(tail_half, tabt_ref, out_ref):
    dim, bm = tabt_ref.shape
    half = bm // 2
    i = pl.program_id(0)
    nblk = pl.num_programs(0)
    xt = tabt_ref[...].T
    hi = jnp.where(i == nblk - 1, xt[tail_half:tail_half + half],
                   xt[half:])
    out_ref[:, pl.ds(0, dim)] = xt[:half]
    out_ref[:, pl.ds(dim, dim)] = hi


@functools.lru_cache(maxsize=None)
def _make_relayout(vocab, dim, bm):
    tail = vocab % bm
    tail_half = (tail // 2) if tail else (bm // 2)
    return pl.pallas_call(
        functools.partial(_relayout_body, tail_half),
        grid=((vocab + bm - 1) // bm,),
        in_specs=[pl.BlockSpec((dim, bm), lambda i: (0, i))],
        out_specs=pl.BlockSpec((bm // 2, 2 * dim), lambda i: (i, 0)),
        out_shape=jax.ShapeDtypeStruct((vocab // 2, 2 * dim), jnp.float32),
        compiler_params=pltpu.CompilerParams(
            dimension_semantics=("arbitrary",)),
    )


def _gather_body(idx_hbm, tab_hbm, out_hbm, idx_v, buf0, buf1, gs0, gs1, ws0, ws1):
    n_chunks = idx_v.shape[0]
    dim = tab_hbm.shape[1]
    half_batch = out_hbm.shape[1]
    batch = 2 * half_batch
    wid = lax.axis_index("s") * _NC + lax.axis_index("c")
    pltpu.sync_copy(idx_hbm.at[wid], idx_v)
    flat0 = wid * (n_chunks * _CHUNK)

    def g_copy(buf, sem, chunk):
        return pltpu.make_async_copy(tab_hbm.at[idx_v.at[chunk]], buf, sem)

    def w_copy(buf, sem, chunk):
        fl = flat0 + chunk * _CHUNK
        l = fl // batch
        bb = fl % batch
        dst = out_hbm.at[l, pl.ds(bb % half_batch, _CHUNK),
                         pl.ds((bb // half_batch) * dim, dim)]
        return pltpu.make_async_copy(buf, dst, sem)

    g_copy(buf0, gs0, 0).start()
    g_copy(buf1, gs1, 1).start()

    def body(i, carry):
        c0 = 2 * i
        g_copy(buf0, gs0, c0).wait()
        w_copy(buf0, ws0, c0).start()
        g_copy(buf1, gs1, c0 + 1).wait()
        w_copy(buf1, ws1, c0 + 1).start()
        w_copy(buf0, ws0, c0).wait()
        g_copy(buf0, gs0, c0 + 2).start()
        w_copy(buf1, ws1, c0 + 1).wait()
        g_copy(buf1, gs1, c0 + 3).start()
        return carry

    lax.fori_loop(0, n_chunks // 2 - 1, body, 0)
    cl = n_chunks - 2
    g_copy(buf0, gs0, cl).wait()
    w_copy(buf0, ws0, cl).start()
    g_copy(buf1, gs1, cl + 1).wait()
    w_copy(buf1, ws1, cl + 1).start()
    w_copy(buf0, ws0, cl).wait()
    w_copy(buf1, ws1, cl + 1).wait()


@functools.lru_cache(maxsize=None)
def _make_gather(hist, half_batch, n_chunks, dim):
    return functools.partial(
        pl.kernel,
        mesh=plsc.VectorSubcoreMesh(core_axis_name="c", subcore_axis_name="s"),
        out_type=jax.ShapeDtypeStruct((hist, half_batch, 2 * dim), jnp.float32),
        scratch_types=[
            pltpu.VMEM((n_chunks, _CHUNK), jnp.int32),
            pltpu.VMEM((_CHUNK, dim), jnp.float32),
            pltpu.VMEM((_CHUNK, dim), jnp.float32),
            pltpu.SemaphoreType.DMA,
            pltpu.SemaphoreType.DMA,
            pltpu.SemaphoreType.DMA,
            pltpu.SemaphoreType.DMA,
        ],
        compiler_params=pltpu.CompilerParams(use_tc_tiling_on_sc=False),
    )(_gather_body)


def _linear_body(*refs):
    if len(refs) == 5:
        emb_ref, w_ref, b_ref, _, out_ref = refs
    else:
        emb_ref, w_ref, b_ref, out_ref = refs
    j = pl.program_id(1)
    hist, dim = emb_ref.shape[0], w_ref.shape[1]
    wt = w_ref[...]
    m0 = (j == 0).astype(jnp.float32)
    w2 = jnp.concatenate([wt * m0, wt * (1.0 - m0)], axis=1)
    bcol = b_ref[...]
    for l in range(hist):
        y = lax.dot_general(w2, emb_ref[l], (((1,), (1,)), ((), ())),
                            preferred_element_type=jnp.float32)
        out_ref[pl.ds(l * dim, dim), :] = jnp.maximum(y + bcol, 0.0)


@functools.lru_cache(maxsize=None)
def _make_linear(batch, hist, dim, out_dim, nbb, span, col_blk0, aliased):
    nblk = (span // 2) // nbb
    in_specs = [
        pl.BlockSpec((hist, nbb, 2 * dim), lambda i, j: (0, i, 0)),
        pl.BlockSpec((out_dim, dim), lambda i, j: (0, 0)),
        pl.BlockSpec((out_dim, 1), lambda i, j: (0, 0)),
    ]
    if aliased:
        in_specs.append(pl.BlockSpec((8, 128), lambda i, j: (0, 0)))
    return pl.pallas_call(
        _linear_body,
        grid=(nblk, 2),
        in_specs=in_specs,
        out_specs=pl.BlockSpec((hist * out_dim, nbb),
                               lambda i, j: (0, col_blk0 + j * nblk + i)),
        out_shape=jax.ShapeDtypeStruct((hist * out_dim, batch), jnp.float32),
        input_output_aliases={3: 0} if aliased else {},
        compiler_params=pltpu.CompilerParams(
            dimension_semantics=("arbitrary", "arbitrary")),
    )


def kernel(text, table, W, b):
    batch, hist = text.shape
    vocab, dim = table.shape
    out_dim = W.shape[0]
    n_rows = batch * hist
    n_chunks = n_rows // (_NW * _CHUNK)
    bm = 32768
    full = (vocab // bm) * bm
    ht = max((vocab - full) // 2, 1)
    q = jnp.bitwise_and(text, bm - 1)
    pseudo_full = jnp.bitwise_and(text, ~jnp.int32(bm - 1)) + 2 * q \
        - jnp.where(q < bm // 2, 0, bm - 1).astype(jnp.int32)
    qt = text - full
    pseudo_tail = full + 2 * qt \
        - jnp.where(qt < ht, 0, 2 * ht - 1).astype(jnp.int32)
    pseudo = jnp.where(text < full, pseudo_full, pseudo_tail)
    pseudo_t = pseudo.T.reshape(hist, 2, batch // 2)
    half = batch // 2
    nc2 = n_chunks // 2
    nbb = 512
    bcol = b.reshape(out_dim, 1)
    tab_pairs = _make_relayout(vocab, dim, 32768)(table.T)
    tab_lin = tab_pairs.reshape(vocab, dim)
    idx_a = pseudo_t[:, 0].reshape(_NW, nc2, _CHUNK)
    idx_b = pseudo_t[:, 1].reshape(_NW, nc2, _CHUNK)
    emb_a = _make_gather(hist, half // 2, nc2, dim)(idx_a, tab_lin)
    out2a = _make_linear(batch, hist, dim, out_dim, nbb, half, 0, False)(
        emb_a, W, bcol)
    emb_b = _make_gather(hist, half // 2, nc2, dim)(idx_b, tab_lin)
    out2 = _make_linear(batch, hist, dim, out_dim, nbb, half,
                        half // nbb, True)(emb_b, W, bcol, out2a)
    return out2.reshape(hist, out_dim, batch).transpose(2, 0, 1)


# speedup vs baseline: 4.1709x; 1.0378x over previous
import functools

import jax
import jax.numpy as jnp
from jax import lax
from jax.experimental import pallas as pl
from jax.experimental.pallas import tpu as pltpu
from jax.experimental.pallas import tpu_sc as plsc

_NC = 2
_NS = 16
_NW = _NC * _NS
_CHUNK = 128


def _relayout_body(tail_half, tabt_ref, out_ref):
    dim, bm = tabt_ref.shape
    half = bm // 2
    i = pl.program_id(0)
    nblk = pl.num_programs(0)
    xt = tabt_ref[...].T
    hi = jnp.where(i == nblk - 1, xt[tail_half:tail_half + half],
                   xt[half:])
    out_ref[:, pl.ds(0, dim)] = xt[:half]
    out_ref[:, pl.ds(dim, dim)] = hi


@functools.lru_cache(maxsize=None)
def _make_relayout(vocab, dim, bm):
    tail = vocab % bm
    tail_half = (tail // 2) if tail else (bm // 2)
    return pl.pallas_call(
        functools.partial(_relayout_body, tail_half),
        grid=((vocab + bm - 1) // bm,),
        in_specs=[pl.BlockSpec((dim, bm), lambda i: (0, i))],
        out_specs=pl.BlockSpec((bm // 2, 2 * dim), lambda i: (i, 0)),
        out_shape=jax.ShapeDtypeStruct((vocab // 2, 2 * dim), jnp.float32),
        compiler_params=pltpu.CompilerParams(
            dimension_semantics=("arbitrary",)),
    )


_NBUF = 4


def _gather_body(idx_hbm, tab_hbm, out_hbm, idx_v, *bufs_sems):
    bufs = bufs_sems[:_NBUF]
    gsems = bufs_sems[_NBUF:2 * _NBUF]
    wsems = bufs_sems[2 * _NBUF:3 * _NBUF]
    n_chunks = idx_v.shape[0]
    dim = tab_hbm.shape[1]
    half_batch = out_hbm.shape[1]
    batch = 2 * half_batch
    wid = lax.axis_index("s") * _NC + lax.axis_index("c")
    pltpu.sync_copy(idx_hbm.at[wid], idx_v)
    flat0 = wid * (n_chunks * _CHUNK)

    def g_copy(k, chunk):
        return pltpu.make_async_copy(tab_hbm.at[idx_v.at[chunk]],
                                     bufs[k], gsems[k])

    def w_copy(k, chunk):
        fl = flat0 + chunk * _CHUNK
        l = fl // batch
        bb = fl % batch
        dst = out_hbm.at[l, pl.ds(bb % half_batch, _CHUNK),
                         pl.ds((bb // half_batch) * dim, dim)]
        return pltpu.make_async_copy(bufs[k], dst, wsems[k])

    for k in range(_NBUF):
        g_copy(k, k).start()

    def body(i, carry):
        c0 = _NBUF * i
        for k in range(_NBUF):
            g_copy(k, c0 + k).wait()
            w_copy(k, c0 + k).start()
        for k in range(_NBUF):
            w_copy(k, c0 + k).wait()
            g_copy(k, c0 + k + _NBUF).start()
        return carry

    lax.fori_loop(0, n_chunks // _NBUF - 1, body, 0)
    cl = n_chunks - _NBUF
    for k in range(_NBUF):
        g_copy(k, cl + k).wait()
        w_copy(k, cl + k).start()
    for k in range(_NBUF):
        w_copy(k, cl + k).wait()


@functools.lru_cache(maxsize=None)
def _make_gather(hist, half_batch, n_chunks, dim):
    return functools.partial(
        pl.kernel,
        mesh=plsc.VectorSubcoreMesh(core_axis_name="c", subcore_axis_name="s"),
        out_type=jax.ShapeDtypeStruct((hist, half_batch, 2 * dim), jnp.float32),
        scratch_types=(
            [pltpu.VMEM((n_chunks, _CHUNK), jnp.int32)]
            + [pltpu.VMEM((_CHUNK, dim), jnp.float32)] * _NBUF
            + [pltpu.SemaphoreType.DMA] * (2 * _NBUF)
        ),
        compiler_params=pltpu.CompilerParams(use_tc_tiling_on_sc=False),
    )(_gather_body)


def _linear_body(*refs):
    if len(refs) == 5:
        emb_ref, w_ref, b_ref, _, out_ref = refs
    else:
        emb_ref, w_ref, b_ref, out_ref = refs
    j = pl.program_id(1)
    hist, dim = emb_ref.shape[0], w_ref.shape[1]
    wt = w_ref[...]
    m0 = (j == 0).astype(jnp.float32)
    w2 = jnp.concatenate([wt * m0, wt * (1.0 - m0)], axis=1)
    bcol = b_ref[...]
    for l in range(hist):
        y = lax.dot_general(w2, emb_ref[l], (((1,), (1,)), ((), ())),
                            preferred_element_type=jnp.float32)
        out_ref[pl.ds(l * dim, dim), :] = jnp.maximum(y + bcol, 0.0)


@functools.lru_cache(maxsize=None)
def _make_linear(batch, hist, dim, out_dim, nbb, span, col_blk0, aliased):
    nblk = (span // 2) // nbb
    in_specs = [
        pl.BlockSpec((hist, nbb, 2 * dim), lambda i, j: (0, i, 0)),
        pl.BlockSpec((out_dim, dim), lambda i, j: (0, 0)),
        pl.BlockSpec((out_dim, 1), lambda i, j: (0, 0)),
    ]
    if aliased:
        in_specs.append(pl.BlockSpec((8, 128), lambda i, j: (0, 0)))
    return pl.pallas_call(
        _linear_body,
        grid=(nblk, 2),
        in_specs=in_specs,
        out_specs=pl.BlockSpec((hist * out_dim, nbb),
                               lambda i, j: (0, col_blk0 + j * nblk + i)),
        out_shape=jax.ShapeDtypeStruct((hist * out_dim, batch), jnp.float32),
        input_output_aliases={3: 0} if aliased else {},
        compiler_params=pltpu.CompilerParams(
            dimension_semantics=("arbitrary", "arbitrary")),
    )


def kernel(text, table, W, b):
    batch, hist = text.shape
    vocab, dim = table.shape
    out_dim = W.shape[0]
    n_rows = batch * hist
    n_chunks = n_rows // (_NW * _CHUNK)
    bm = 32768
    full = (vocab // bm) * bm
    ht = max((vocab - full) // 2, 1)
    q = jnp.bitwise_and(text, bm - 1)
    pseudo_full = jnp.bitwise_and(text, ~jnp.int32(bm - 1)) + 2 * q \
        - jnp.where(q < bm // 2, 0, bm - 1).astype(jnp.int32)
    qt = text - full
    pseudo_tail = full + 2 * qt \
        - jnp.where(qt < ht, 0, 2 * ht - 1).astype(jnp.int32)
    pseudo = jnp.where(text < full, pseudo_full, pseudo_tail)
    pseudo_t = pseudo.T.reshape(hist, 2, batch // 2)
    half = batch // 2
    nc2 = n_chunks // 2
    nbb = 512
    bcol = b.reshape(out_dim, 1)
    tab_pairs = _make_relayout(vocab, dim, 32768)(table.T)
    tab_lin = tab_pairs.reshape(vocab, dim)
    idx_a = pseudo_t[:, 0].reshape(_NW, nc2, _CHUNK)
    idx_b = pseudo_t[:, 1].reshape(_NW, nc2, _CHUNK)
    emb_a = _make_gather(hist, half // 2, nc2, dim)(idx_a, tab_lin)
    out2a = _make_linear(batch, hist, dim, out_dim, nbb, half, 0, False)(
        emb_a, W, bcol)
    emb_b = _make_gather(hist, half // 2, nc2, dim)(idx_b, tab_lin)
    out2 = _make_linear(batch, hist, dim, out_dim, nbb, half,
                        half // nbb, True)(emb_b, W, bcol, out2a)
    return out2.reshape(hist, out_dim, batch).transpose(2, 0, 1)
